# baseline (speedup 1.0000x reference)
import functools

import jax
import jax.numpy as jnp
from jax import lax
from jax.experimental import pallas as pl
from jax.experimental.pallas import tpu as pltpu
from jax.experimental.pallas import tpu_sc as plsc

VOCAB = 1_000_000
EMB = 64
ROWS = 4096
COLS = 200
TOTAL = ROWS * COLS

_info = plsc.get_sparse_core_info()
NC = _info.num_cores
NS = _info.num_subcores
NW = NC * NS
B_PER_W = TOTAL // NW
CHUNK = 128
CHUNKS = B_PER_W // CHUNK

_mesh = plsc.VectorSubcoreMesh(core_axis_name="c", subcore_axis_name="s")


@functools.partial(
    pl.kernel,
    out_type=jax.ShapeDtypeStruct((TOTAL, EMB), jnp.float32),
    mesh=_mesh,
    scratch_types=[
        pltpu.VMEM((CHUNKS, CHUNK), jnp.int32),
        pltpu.VMEM((CHUNK, EMB), jnp.float32),
        pltpu.SemaphoreType.DMA,
    ],
    compiler_params=pltpu.CompilerParams(use_tc_tiling_on_sc=False),
)
def _embed_sc(idx_hbm, table_hbm, out_hbm, idx_v, rows_v, gsem):
    wid = lax.axis_index("s") * NC + lax.axis_index("c")
    base = wid * B_PER_W
    pltpu.sync_copy(idx_hbm.at[wid], idx_v)

    def step(j, carry):
        pltpu.async_copy(table_hbm.at[idx_v.at[j]], rows_v, gsem).wait()
        pltpu.sync_copy(rows_v, out_hbm.at[pl.ds(base + j * CHUNK, CHUNK)])
        return carry

    lax.fori_loop(0, CHUNKS, step, 0)


def kernel(input_tensor, weight):
    idx = input_tensor.reshape(NW, CHUNKS, CHUNK).astype(jnp.int32)
    out = _embed_sc(idx, weight)
    return out.reshape(ROWS, COLS, EMB)


# speedup vs baseline: 1.1097x; 1.1097x over previous
import functools

import jax
import jax.numpy as jnp
from jax import lax
from jax.experimental import pallas as pl
from jax.experimental.pallas import tpu as pltpu
from jax.experimental.pallas import tpu_sc as plsc

VOCAB = 1_000_000
EMB = 64
ROWS = 4096
COLS = 200
TOTAL = ROWS * COLS

_info = plsc.get_sparse_core_info()
NC = _info.num_cores
NS = _info.num_subcores
NW = NC * NS
B_PER_W = TOTAL // NW
CHUNK = 128
CHUNKS = B_PER_W // CHUNK
NBUF = 4

_mesh = plsc.VectorSubcoreMesh(core_axis_name="c", subcore_axis_name="s")


@functools.partial(
    pl.kernel,
    out_type=jax.ShapeDtypeStruct((TOTAL, EMB), jnp.float32),
    mesh=_mesh,
    scratch_types=[
        pltpu.VMEM((CHUNKS, CHUNK), jnp.int32),
        pltpu.VMEM((NBUF, CHUNK, EMB), jnp.float32),
        pltpu.SemaphoreType.DMA((NBUF,)),
        pltpu.SemaphoreType.DMA((NBUF,)),
    ],
    compiler_params=pltpu.CompilerParams(use_tc_tiling_on_sc=False),
)
def _embed_sc(idx_hbm, table_hbm, out_hbm, idx_v, rows_v, gsem, ssem):
    wid = lax.axis_index("s") * NC + lax.axis_index("c")
    base = wid * B_PER_W
    pltpu.sync_copy(idx_hbm.at[wid], idx_v)

    def fire_gather(j, b):
        pltpu.async_copy(table_hbm.at[idx_v.at[j]], rows_v.at[b], gsem.at[b])

    def wait_gather(b):
        pltpu.make_async_copy(out_hbm.at[pl.ds(0, CHUNK)], rows_v.at[b],
                              gsem.at[b]).wait()

    def fire_store(j, b):
        pltpu.async_copy(rows_v.at[b],
                         out_hbm.at[pl.ds(base + j * CHUNK, CHUNK)],
                         ssem.at[b])

    def wait_store(b):
        pltpu.make_async_copy(rows_v.at[b], out_hbm.at[pl.ds(0, CHUNK)],
                              ssem.at[b]).wait()

    for b in range(NBUF):
        fire_gather(b, b)

    n_rounds = CHUNKS // NBUF

    def round_body(t, carry):
        j0 = t * NBUF
        for b in range(NBUF):
            wait_gather(b)
            fire_store(j0 + b, b)
        for b in range(NBUF):
            wait_store(b)
            fire_gather(j0 + NBUF + b, b)
        return carry

    lax.fori_loop(0, n_rounds - 1, round_body, 0)

    j0 = (n_rounds - 1) * NBUF
    for b in range(NBUF):
        wait_gather(b)
        fire_store(j0 + b, b)
    for b in range(NBUF):
        wait_store(b)


def kernel(input_tensor, weight):
    idx = input_tensor.reshape(NW, CHUNKS, CHUNK).astype(jnp.int32)
    out = _embed_sc(idx, weight)
    return out.reshape(ROWS, COLS, EMB)
